# Initial kernel scaffold; baseline (speedup 1.0000x reference)
import jax
import jax.numpy as jnp
from jax.experimental import pallas as pl


def kernel(x, pos, batch, W_lin, W_src, W_dst, W_pos1, b_pos1, W_pos2, b_pos2, W_att, b_att, gamma, beta):
    raise NotImplementedError("write your pallas kernel here")


# speedup vs baseline: 2.6194x; 2.6194x over previous
import functools

import jax
import jax.numpy as jnp
from jax import lax
from jax.experimental import pallas as pl
from jax.experimental.pallas import tpu as pltpu
from jax.experimental.pallas import tpu_sc as plsc

N = 10000
D = 128
K = 16
R2 = 0.25
B = 8
NEG = -1e30
BIG = 1e30

NP = 10240
BLK_A = 64
BLK_B = 64
E = NP * K


def _knn_body(posr_ref, posc_ref, batr_ref, batc_ref, wp1_ref,
              nbr_ref, val_ref, q_ref):
    pr = posr_ref[...]
    pc = posc_ref[...]
    br = batr_ref[...]
    bc = batc_ref[...]

    q_ref[...] = jnp.dot(pr, wp1_ref[...], preferred_element_type=jnp.float32)

    d2 = ((pr[:, 0:1] - pc[0:1, :]) ** 2
          + (pr[:, 1:2] - pc[1:2, :]) ** 2
          + (pr[:, 2:3] - pc[2:3, :]) ** 2)
    ok = (br == bc) & (d2 <= R2)
    d = jnp.where(ok, d2, BIG)

    cols = lax.broadcasted_iota(jnp.int32, (BLK_A, NP), 1)
    for k in range(K):
        m = jnp.min(d, axis=1, keepdims=True)
        cand = jnp.where(d == m, cols, jnp.int32(NP))
        idx = jnp.min(cand, axis=1, keepdims=True)
        d = jnp.where(cols == idx, BIG, d)
        good = m <= R2
        nbr_ref[:, k:k + 1] = jnp.where(good, idx, 0)
        val_ref[:, k:k + 1] = jnp.where(good, 1.0, 0.0)


def _knn(posr, posc, batr, batc, W_pos1):
    grid = NP // BLK_A
    return pl.pallas_call(
        _knn_body,
        grid=(grid,),
        in_specs=[
            pl.BlockSpec((BLK_A, 3), lambda i: (i, 0)),
            pl.BlockSpec((3, NP), lambda i: (0, 0)),
            pl.BlockSpec((BLK_A, 1), lambda i: (i, 0)),
            pl.BlockSpec((1, NP), lambda i: (0, 0)),
            pl.BlockSpec((3, D), lambda i: (0, 0)),
        ],
        out_specs=[
            pl.BlockSpec((BLK_A, K), lambda i: (i, 0)),
            pl.BlockSpec((BLK_A, K), lambda i: (i, 0)),
            pl.BlockSpec((BLK_A, D), lambda i: (i, 0)),
        ],
        out_shape=[
            jax.ShapeDtypeStruct((NP, K), jnp.int32),
            jax.ShapeDtypeStruct((NP, K), jnp.float32),
            jax.ShapeDtypeStruct((NP, D), jnp.float32),
        ],
    )(posr, posc, batr, batc, W_pos1)


_SC_CHUNK = 256


def _sc_gather(idx_flat, x, q):
    info = plsc.get_sparse_core_info()
    nw = info.num_cores * info.num_subcores
    epw = E // nw
    nch = epw // _SC_CHUNK
    mesh = plsc.VectorSubcoreMesh(core_axis_name="c", subcore_axis_name="s")

    @functools.partial(
        pl.kernel,
        out_type=[
            jax.ShapeDtypeStruct((E, D), jnp.float32),
            jax.ShapeDtypeStruct((E, D), jnp.float32),
        ],
        mesh=mesh,
        scratch_types=[
            pltpu.VMEM((_SC_CHUNK,), jnp.int32),
            pltpu.VMEM((_SC_CHUNK, D), jnp.float32),
            pltpu.VMEM((_SC_CHUNK, D), jnp.float32),
            pltpu.SemaphoreType.DMA,
            pltpu.SemaphoreType.DMA,
        ],
    )
    def gather(idx_hbm, x_hbm, q_hbm, xg_out, qg_out,
               idx_v, xrow_v, qrow_v, semx, semq):
        wid = lax.axis_index("s") * info.num_cores + lax.axis_index("c")
        for c in range(nch):
            base = wid * epw + c * _SC_CHUNK
            pltpu.sync_copy(idx_hbm.at[pl.ds(base, _SC_CHUNK)], idx_v)
            cpx = pltpu.async_copy(x_hbm.at[idx_v], xrow_v, semx)
            cpq = pltpu.async_copy(q_hbm.at[idx_v], qrow_v, semq)
            cpx.wait()
            cpq.wait()
            pltpu.sync_copy(xrow_v, xg_out.at[pl.ds(base, _SC_CHUNK)])
            pltpu.sync_copy(qrow_v, qg_out.at[pl.ds(base, _SC_CHUNK)])

    return gather(idx_flat, x, q)


def _block_body(x_ref, q_ref, xg_ref, qg_ref, val_ref,
                wlin_ref, wsrc_ref, wdst_ref, bp1_ref,
                wp2_ref, bp2_ref, watt_ref, batt_ref, g_ref, b_ref,
                out_ref):
    f32 = jnp.float32
    xg = xg_ref[...]
    a_src = jnp.dot(xg, wsrc_ref[...], preferred_element_type=f32)
    v_e = jnp.dot(xg, wlin_ref[...], preferred_element_type=f32)

    q = q_ref[...]
    qg = qg_ref[...].reshape(BLK_B, K, D)
    h = jnp.maximum(
        (q[:, None, :] - qg).reshape(BLK_B * K, D) + bp1_ref[...], 0.0)
    delta = jnp.maximum(
        jnp.dot(h, wp2_ref[...], preferred_element_type=f32) + bp2_ref[...],
        0.0)

    a_dst = jnp.dot(x_ref[...], wdst_ref[...], preferred_element_type=f32)
    ai = (a_dst[:, None, :] - a_src.reshape(BLK_B, K, D)
          + delta.reshape(BLK_B, K, D)).reshape(BLK_B * K, D)
    alpha = jnp.maximum(
        jnp.dot(ai, watt_ref[...], preferred_element_type=f32) + batt_ref[...],
        0.0)

    v3 = val_ref[...][:, :, None] > 0.0
    al3 = jnp.where(v3, alpha.reshape(BLK_B, K, D), NEG)
    mx = jnp.max(al3, axis=1, keepdims=True)
    ex = jnp.exp(al3 - mx)
    sm = ex / jnp.sum(ex, axis=1, keepdims=True)
    sm = jnp.where(v3, sm, 0.0)

    msg = sm * (v_e.reshape(BLK_B, K, D) + delta.reshape(BLK_B, K, D))
    out = jnp.sum(msg, axis=1)

    mu = jnp.mean(out, axis=-1, keepdims=True)
    var = jnp.mean((out - mu) ** 2, axis=-1, keepdims=True)
    y = (out - mu) / jnp.sqrt(var + 1e-5)
    out_ref[...] = y * g_ref[...] + b_ref[...]


def _block(x_p, q_p, xg, qg, validf, W_lin, W_src, W_dst,
           b_pos1, W_pos2, b_pos2, W_att, b_att, gamma, beta):
    grid = NP // BLK_B
    full = lambda r, c: pl.BlockSpec((r, c), lambda i: (0, 0))
    return pl.pallas_call(
        _block_body,
        grid=(grid,),
        in_specs=[
            pl.BlockSpec((BLK_B, D), lambda i: (i, 0)),
            pl.BlockSpec((BLK_B, D), lambda i: (i, 0)),
            pl.BlockSpec((BLK_B * K, D), lambda i: (i, 0)),
            pl.BlockSpec((BLK_B * K, D), lambda i: (i, 0)),
            pl.BlockSpec((BLK_B, K), lambda i: (i, 0)),
            full(D, D), full(D, D), full(D, D),
            full(1, D),
            full(D, D), full(1, D),
            full(D, D), full(1, D),
            full(1, D), full(1, D),
        ],
        out_specs=pl.BlockSpec((BLK_B, D), lambda i: (i, 0)),
        out_shape=jax.ShapeDtypeStruct((NP, D), jnp.float32),
    )(x_p, q_p, xg, qg, validf, W_lin, W_src, W_dst,
      b_pos1, W_pos2, b_pos2, W_att, b_att, gamma, beta)


def kernel(x, pos, batch, W_lin, W_src, W_dst, W_pos1, b_pos1,
           W_pos2, b_pos2, W_att, b_att, gamma, beta):
    batch = batch.astype(jnp.int32)

    posr = jnp.pad(pos, ((0, NP - N), (0, 0)))
    posc = posr.T
    batr = jnp.pad(batch, (0, NP - N), constant_values=-1).reshape(NP, 1)
    batc = jnp.pad(batch, (0, NP - N), constant_values=-2).reshape(1, NP)

    nbr, validf, q_p = _knn(posr, posc, batr, batc, W_pos1)

    xg, qg = _sc_gather(nbr.reshape(E), x, q_p[:N])

    x_p = jnp.pad(x, ((0, NP - N), (0, 0)))

    y = _block(x_p, q_p, xg, qg, validf,
               W_lin, W_src, W_dst, b_pos1.reshape(1, D),
               W_pos2, b_pos2.reshape(1, D), W_att, b_att.reshape(1, D),
               gamma.reshape(1, D), beta.reshape(1, D))

    return y[:N], pos, batch


# speedup vs baseline: 2.6228x; 1.0013x over previous
import functools

import jax
import jax.numpy as jnp
from jax import lax
from jax.experimental import pallas as pl
from jax.experimental.pallas import tpu as pltpu
from jax.experimental.pallas import tpu_sc as plsc

N = 10000
D = 128
K = 16
R2 = 0.25
B = 8
NEG = -1e30
BIG = 1e30

NP = 10240
BLK_A = 64
BLK_B = 64
E = NP * K


def _knn_body(posr_ref, posc_ref, batr_ref, batc_ref, wp1_ref,
              nbr_ref, val_ref, q_ref):
    pr = posr_ref[...]
    pc = posc_ref[...]
    br = batr_ref[...]
    bc = batc_ref[...]

    q_ref[...] = jnp.dot(pr, wp1_ref[...], preferred_element_type=jnp.float32)

    d2 = ((pr[:, 0:1] - pc[0:1, :]) ** 2
          + (pr[:, 1:2] - pc[1:2, :]) ** 2
          + (pr[:, 2:3] - pc[2:3, :]) ** 2)
    ok = (br == bc) & (d2 <= R2)
    d = jnp.where(ok, d2, BIG)

    cols = lax.broadcasted_iota(jnp.int32, (BLK_A, NP), 1)
    for k in range(K):
        m = jnp.min(d, axis=1, keepdims=True)
        cand = jnp.where(d == m, cols, jnp.int32(NP))
        idx = jnp.min(cand, axis=1, keepdims=True)
        d = jnp.where(cols == idx, BIG, d)
        good = m <= R2
        nbr_ref[:, k:k + 1] = jnp.where(good, idx, 0)
        val_ref[:, k:k + 1] = jnp.where(good, 1.0, 0.0)


def _knn(posr, posc, batr, batc, W_pos1):
    grid = NP // BLK_A
    return pl.pallas_call(
        _knn_body,
        grid=(grid,),
        in_specs=[
            pl.BlockSpec((BLK_A, 3), lambda i: (i, 0)),
            pl.BlockSpec((3, NP), lambda i: (0, 0)),
            pl.BlockSpec((BLK_A, 1), lambda i: (i, 0)),
            pl.BlockSpec((1, NP), lambda i: (0, 0)),
            pl.BlockSpec((3, D), lambda i: (0, 0)),
        ],
        out_specs=[
            pl.BlockSpec((BLK_A, K), lambda i: (i, 0)),
            pl.BlockSpec((BLK_A, K), lambda i: (i, 0)),
            pl.BlockSpec((BLK_A, D), lambda i: (i, 0)),
        ],
        out_shape=[
            jax.ShapeDtypeStruct((NP, K), jnp.int32),
            jax.ShapeDtypeStruct((NP, K), jnp.float32),
            jax.ShapeDtypeStruct((NP, D), jnp.float32),
        ],
    )(posr, posc, batr, batc, W_pos1)


_SC_CHUNK = 128
_SC_NBUF = 3


def _sc_gather(idx_flat, x, q):
    info = plsc.get_sparse_core_info()
    nw = info.num_cores * info.num_subcores
    epw = E // nw
    nch = epw // _SC_CHUNK
    mesh = plsc.VectorSubcoreMesh(core_axis_name="c", subcore_axis_name="s")

    @functools.partial(
        pl.kernel,
        out_type=[
            jax.ShapeDtypeStruct((E, D), jnp.float32),
            jax.ShapeDtypeStruct((E, D), jnp.float32),
        ],
        mesh=mesh,
        scratch_types=[
            pltpu.VMEM((epw,), jnp.int32),
            pltpu.VMEM((_SC_NBUF, _SC_CHUNK, D), jnp.float32),
            pltpu.VMEM((_SC_NBUF, _SC_CHUNK, D), jnp.float32),
        ] + [pltpu.SemaphoreType.DMA] * (2 * _SC_NBUF),
    )
    def gather(idx_hbm, x_hbm, q_hbm, xg_out, qg_out,
               idx_v, xbuf, qbuf, *sems):
        gsem = sems[:_SC_NBUF]
        wsem = sems[_SC_NBUF:]
        wid = lax.axis_index("s") * info.num_cores + lax.axis_index("c")
        w0 = wid * epw
        pltpu.sync_copy(idx_hbm.at[pl.ds(w0, epw)], idx_v)

        g = {}
        wb = {}

        def flush(c):
            g[c][0].wait()
            g[c][1].wait()
            b = c % _SC_NBUF
            base = w0 + c * _SC_CHUNK
            wb[c] = (
                pltpu.async_copy(xbuf.at[b], xg_out.at[pl.ds(base, _SC_CHUNK)],
                                 wsem[b]),
                pltpu.async_copy(qbuf.at[b], qg_out.at[pl.ds(base, _SC_CHUNK)],
                                 wsem[b]),
            )

        for c in range(nch):
            b = c % _SC_NBUF
            if c >= _SC_NBUF:
                wb[c - _SC_NBUF][0].wait()
                wb[c - _SC_NBUF][1].wait()
            isl = idx_v.at[pl.ds(c * _SC_CHUNK, _SC_CHUNK)]
            g[c] = (
                pltpu.async_copy(x_hbm.at[isl], xbuf.at[b], gsem[b]),
                pltpu.async_copy(q_hbm.at[isl], qbuf.at[b], gsem[b]),
            )
            if c >= 1:
                flush(c - 1)
        flush(nch - 1)
        for c in range(nch - _SC_NBUF, nch):
            wb[c][0].wait()
            wb[c][1].wait()

    return gather(idx_flat, x, q)


def _block_body(x_ref, q_ref, xg_ref, qg_ref, val_ref,
                wlin_ref, wsrc_ref, wdst_ref, bp1_ref,
                wp2_ref, bp2_ref, watt_ref, batt_ref, g_ref, b_ref,
                out_ref):
    f32 = jnp.float32
    xg = xg_ref[...]
    a_src = jnp.dot(xg, wsrc_ref[...], preferred_element_type=f32)
    v_e = jnp.dot(xg, wlin_ref[...], preferred_element_type=f32)

    q = q_ref[...]
    qg = qg_ref[...].reshape(BLK_B, K, D)
    h = jnp.maximum(
        (q[:, None, :] - qg).reshape(BLK_B * K, D) + bp1_ref[...], 0.0)
    delta = jnp.maximum(
        jnp.dot(h, wp2_ref[...], preferred_element_type=f32) + bp2_ref[...],
        0.0)

    a_dst = jnp.dot(x_ref[...], wdst_ref[...], preferred_element_type=f32)
    ai = (a_dst[:, None, :] - a_src.reshape(BLK_B, K, D)
          + delta.reshape(BLK_B, K, D)).reshape(BLK_B * K, D)
    alpha = jnp.maximum(
        jnp.dot(ai, watt_ref[...], preferred_element_type=f32) + batt_ref[...],
        0.0)

    v3 = val_ref[...][:, :, None] > 0.0
    al3 = jnp.where(v3, alpha.reshape(BLK_B, K, D), NEG)
    mx = jnp.max(al3, axis=1, keepdims=True)
    ex = jnp.exp(al3 - mx)
    sm = ex / jnp.sum(ex, axis=1, keepdims=True)
    sm = jnp.where(v3, sm, 0.0)

    msg = sm * (v_e.reshape(BLK_B, K, D) + delta.reshape(BLK_B, K, D))
    out = jnp.sum(msg, axis=1)

    mu = jnp.mean(out, axis=-1, keepdims=True)
    var = jnp.mean((out - mu) ** 2, axis=-1, keepdims=True)
    y = (out - mu) / jnp.sqrt(var + 1e-5)
    out_ref[...] = y * g_ref[...] + b_ref[...]


def _block(x_p, q_p, xg, qg, validf, W_lin, W_src, W_dst,
           b_pos1, W_pos2, b_pos2, W_att, b_att, gamma, beta):
    grid = NP // BLK_B
    full = lambda r, c: pl.BlockSpec((r, c), lambda i: (0, 0))
    return pl.pallas_call(
        _block_body,
        grid=(grid,),
        in_specs=[
            pl.BlockSpec((BLK_B, D), lambda i: (i, 0)),
            pl.BlockSpec((BLK_B, D), lambda i: (i, 0)),
            pl.BlockSpec((BLK_B * K, D), lambda i: (i, 0)),
            pl.BlockSpec((BLK_B * K, D), lambda i: (i, 0)),
            pl.BlockSpec((BLK_B, K), lambda i: (i, 0)),
            full(D, D), full(D, D), full(D, D),
            full(1, D),
            full(D, D), full(1, D),
            full(D, D), full(1, D),
            full(1, D), full(1, D),
        ],
        out_specs=pl.BlockSpec((BLK_B, D), lambda i: (i, 0)),
        out_shape=jax.ShapeDtypeStruct((NP, D), jnp.float32),
    )(x_p, q_p, xg, qg, validf, W_lin, W_src, W_dst,
      b_pos1, W_pos2, b_pos2, W_att, b_att, gamma, beta)


def kernel(x, pos, batch, W_lin, W_src, W_dst, W_pos1, b_pos1,
           W_pos2, b_pos2, W_att, b_att, gamma, beta):
    batch = batch.astype(jnp.int32)

    posr = jnp.pad(pos, ((0, NP - N), (0, 0)))
    posc = posr.T
    batr = jnp.pad(batch, (0, NP - N), constant_values=-1).reshape(NP, 1)
    batc = jnp.pad(batch, (0, NP - N), constant_values=-2).reshape(1, NP)

    nbr, validf, q_p = _knn(posr, posc, batr, batc, W_pos1)

    xg, qg = _sc_gather(nbr.reshape(E), x, q_p[:N])

    x_p = jnp.pad(x, ((0, NP - N), (0, 0)))

    y = _block(x_p, q_p, xg, qg, validf,
               W_lin, W_src, W_dst, b_pos1.reshape(1, D),
               W_pos2, b_pos2.reshape(1, D), W_att, b_att.reshape(1, D),
               gamma.reshape(1, D), beta.reshape(1, D))

    return y[:N], pos, batch
